# baseline (speedup 1.0000x reference)
import functools

import jax
import jax.numpy as jnp
from jax import lax
from jax.experimental import pallas as pl
from jax.experimental.pallas import tpu as pltpu
from jax.experimental.pallas import tpu_sc as plsc

BATCH = 16384
SEQ = 200
EMBED = 128
PAD_IDX = 2
TOTAL = BATCH * SEQ

NC = 2
NS = 16
NW = NC * NS
PER_W = TOTAL // NW
CHUNK = 128
K = 16
NBLK = PER_W // (CHUNK * K)
L = 16
NV = EMBED // L

_mesh = plsc.VectorSubcoreMesh(core_axis_name="c", subcore_axis_name="s")


@functools.partial(
    pl.kernel,
    out_type=jax.ShapeDtypeStruct((TOTAL, EMBED), jnp.float32),
    mesh=_mesh,
    scratch_types=[
        pltpu.VMEM((3, EMBED), jnp.float32),
        pltpu.VMEM((K, CHUNK), jnp.int32),
        pltpu.VMEM((CHUNK, EMBED), jnp.float32),
        pltpu.VMEM((CHUNK, EMBED), jnp.float32),
        pltpu.SemaphoreType.DMA,
        pltpu.SemaphoreType.DMA,
    ],
)
def _sc_embed(idx_hbm, table_hbm, out_hbm, table_v, idx_v, rows0, rows1,
              ss0, ss1):
    wid = lax.axis_index("s") * NC + lax.axis_index("c")
    crow0 = wid * (PER_W // CHUNK)
    rbase = wid * PER_W
    bufs = (rows0, rows1)
    ssems = (ss0, ss1)

    pltpu.sync_copy(table_hbm, table_v)
    t0 = [table_v[0, pl.ds(k * L, L)] for k in range(NV)]
    t1 = [table_v[1, pl.ds(k * L, L)] for k in range(NV)]
    one = jnp.ones((L,), jnp.int32)
    zero = jnp.zeros((L,), jnp.int32)

    def fill_rows(buf, j):
        def group(g, carry):
            ivec = idx_v[j, pl.ds(g * L, L)]
            w0f = jnp.maximum(zero, one - ivec).astype(jnp.float32)
            w1f = jnp.maximum(zero, one - jnp.abs(ivec - one)).astype(
                jnp.float32)
            for l in range(L):
                a = jnp.full((L,), w0f[l], jnp.float32)
                b = jnp.full((L,), w1f[l], jnp.float32)
                i = g * L + l
                for k in range(NV):
                    buf[i, pl.ds(k * L, L)] = a * t0[k] + b * t1[k]
            return carry
        lax.fori_loop(0, CHUNK // L, group, 0)

    def block(blk, carry):
        pltpu.sync_copy(idx_hbm.at[pl.ds(crow0 + blk * K, K)], idx_v)
        stores = [None, None]
        for j in range(K):
            b = j & 1
            if stores[b] is not None:
                stores[b].wait()
            fill_rows(bufs[b], j)
            stores[b] = pltpu.async_copy(
                bufs[b],
                out_hbm.at[pl.ds(rbase + (blk * K + j) * CHUNK, CHUNK)],
                ssems[b],
            )
        stores[0].wait()
        stores[1].wait()
        return carry

    lax.fori_loop(0, NBLK, block, 0)


def kernel(inputs, table):
    pad_mask = (jnp.arange(3) != PAD_IDX).astype(table.dtype)[:, None]
    masked_table = table * pad_mask
    idx2d = inputs.reshape(TOTAL // CHUNK, CHUNK)
    out = _sc_embed(idx2d, masked_table)
    return out.reshape(BATCH, SEQ, EMBED)


# speedup vs baseline: 1.6336x; 1.6336x over previous
import functools

import jax
import jax.numpy as jnp
from jax import lax
from jax.experimental import pallas as pl
from jax.experimental.pallas import tpu as pltpu
from jax.experimental.pallas import tpu_sc as plsc

BATCH = 16384
SEQ = 200
EMBED = 128
PAD_IDX = 2
TOTAL = BATCH * SEQ

NC = 2
NS = 16
NW = NC * NS
PER_W = TOTAL // NW
CHUNK = 128
K = 16
NBLK = PER_W // (CHUNK * K)
L = 16
NV = EMBED // L

VALU_CHUNK = tuple(j % 3 == 1 for j in range(K))

_mesh = plsc.VectorSubcoreMesh(core_axis_name="c", subcore_axis_name="s")


@functools.partial(
    pl.kernel,
    out_type=jax.ShapeDtypeStruct((TOTAL, EMBED), jnp.float32),
    mesh=_mesh,
    scratch_types=[
        pltpu.VMEM_SHARED((3, EMBED), jnp.float32),
        pltpu.VMEM((3, EMBED), jnp.float32),
        pltpu.VMEM((K, CHUNK), jnp.int32),
        pltpu.VMEM((CHUNK, EMBED), jnp.float32),
        pltpu.VMEM((CHUNK, EMBED), jnp.float32),
        pltpu.VMEM((CHUNK, EMBED), jnp.float32),
        pltpu.VMEM((CHUNK, EMBED), jnp.float32),
        pltpu.SemaphoreType.DMA,
        pltpu.SemaphoreType.DMA,
        pltpu.SemaphoreType.DMA,
        pltpu.SemaphoreType.DMA,
        pltpu.SemaphoreType.DMA,
        pltpu.SemaphoreType.DMA,
    ],
)
def _sc_embed(idx_hbm, table_hbm, out_hbm, table_s, table_v, idx_v,
              gbuf0, gbuf1, vbuf0, vbuf1,
              gs0, gs1, sg0, sg1, sv0, sv1):
    wid = lax.axis_index("s") * NC + lax.axis_index("c")
    crow0 = wid * (PER_W // CHUNK)
    rbase = wid * PER_W
    gbufs = (gbuf0, gbuf1)
    vbufs = (vbuf0, vbuf1)
    gsems = (gs0, gs1)
    sgsems = (sg0, sg1)
    svsems = (sv0, sv1)

    @pl.when(lax.axis_index("s") == 0)
    def _stage_shared_table():
        pltpu.sync_copy(table_hbm, table_s)

    pltpu.sync_copy(table_hbm, table_v)
    plsc.subcore_barrier()

    t0 = [table_v[0, pl.ds(k * L, L)] for k in range(NV)]
    t1 = [table_v[1, pl.ds(k * L, L)] for k in range(NV)]
    one = jnp.ones((L,), jnp.int32)
    zero = jnp.zeros((L,), jnp.int32)

    def fill_rows(buf, j):
        def group(g, carry):
            ivec = idx_v[j, pl.ds(g * L, L)]
            w0f = jnp.maximum(zero, one - ivec).astype(jnp.float32)
            w1f = jnp.maximum(zero, one - jnp.abs(ivec - one)).astype(
                jnp.float32)
            for l in range(L):
                a = jnp.full((L,), w0f[l], jnp.float32)
                b = jnp.full((L,), w1f[l], jnp.float32)
                i = g * L + l
                for k in range(NV):
                    buf[i, pl.ds(k * L, L)] = a * t0[k] + b * t1[k]
            return carry
        lax.fori_loop(0, CHUNK // L, group, 0, unroll=2)

    def block(blk, carry):
        pltpu.sync_copy(idx_hbm.at[pl.ds(crow0 + blk * K, K)], idx_v)
        gathers = [None, None]
        gstores = [None, None]
        vstores = [None, None]
        gcnt = 0
        vcnt = 0
        pending = None
        for j in range(K):
            if VALU_CHUNK[j]:
                b = vcnt & 1
                if vstores[b] is not None:
                    vstores[b].wait()
                fill_rows(vbufs[b], j)
                vstores[b] = pltpu.async_copy(
                    vbufs[b],
                    out_hbm.at[pl.ds(rbase + (blk * K + j) * CHUNK, CHUNK)],
                    svsems[b],
                )
                vcnt += 1
            else:
                b = gcnt & 1
                if gstores[b] is not None:
                    gstores[b].wait()
                gathers[b] = pltpu.async_copy(
                    table_s.at[idx_v.at[j]], gbufs[b], gsems[b])
                if pending is not None:
                    pc, pb, pj = pending
                    pc.wait()
                    gstores[pb] = pltpu.async_copy(
                        gbufs[pb],
                        out_hbm.at[pl.ds(rbase + (blk * K + pj) * CHUNK,
                                         CHUNK)],
                        sgsems[pb],
                    )
                pending = (gathers[b], b, j)
                gcnt += 1
        pc, pb, pj = pending
        pc.wait()
        gstores[pb] = pltpu.async_copy(
            gbufs[pb],
            out_hbm.at[pl.ds(rbase + (blk * K + pj) * CHUNK, CHUNK)],
            sgsems[pb],
        )
        for st in gstores + vstores:
            if st is not None:
                st.wait()
        return carry

    lax.fori_loop(0, NBLK, block, 0)


def kernel(inputs, table):
    pad_mask = (jnp.arange(3) != PAD_IDX).astype(table.dtype)[:, None]
    masked_table = table * pad_mask
    idx2d = inputs.reshape(TOTAL // CHUNK, CHUNK)
    out = _sc_embed(idx2d, masked_table)
    return out.reshape(BATCH, SEQ, EMBED)


# speedup vs baseline: 1.6793x; 1.0279x over previous
import functools

import jax
import jax.numpy as jnp
from jax import lax
from jax.experimental import pallas as pl
from jax.experimental.pallas import tpu as pltpu
from jax.experimental.pallas import tpu_sc as plsc

BATCH = 16384
SEQ = 200
EMBED = 128
PAD_IDX = 2
TOTAL = BATCH * SEQ

NC = 2
NS = 16
NW = NC * NS
PER_W = TOTAL // NW
CHUNK = 128
K = 16
NBLK = PER_W // (CHUNK * K)
L = 16
NV = EMBED // L

VALU_CHUNK = tuple(j in (1, 4, 6, 9, 12, 14) for j in range(K))

_mesh = plsc.VectorSubcoreMesh(core_axis_name="c", subcore_axis_name="s")


@functools.partial(
    pl.kernel,
    out_type=jax.ShapeDtypeStruct((TOTAL, EMBED), jnp.float32),
    mesh=_mesh,
    scratch_types=[
        pltpu.VMEM_SHARED((3, EMBED), jnp.float32),
        pltpu.VMEM((3, EMBED), jnp.float32),
        pltpu.VMEM((K, CHUNK), jnp.int32),
        pltpu.VMEM((CHUNK, EMBED), jnp.float32),
        pltpu.VMEM((CHUNK, EMBED), jnp.float32),
        pltpu.VMEM((CHUNK, EMBED), jnp.float32),
        pltpu.VMEM((CHUNK, EMBED), jnp.float32),
        pltpu.SemaphoreType.DMA,
        pltpu.SemaphoreType.DMA,
        pltpu.SemaphoreType.DMA,
        pltpu.SemaphoreType.DMA,
        pltpu.SemaphoreType.DMA,
        pltpu.SemaphoreType.DMA,
    ],
)
def _sc_embed(idx_hbm, table_hbm, out_hbm, table_s, table_v, idx_v,
              gbuf0, gbuf1, vbuf0, vbuf1,
              gs0, gs1, sg0, sg1, sv0, sv1):
    wid = lax.axis_index("s") * NC + lax.axis_index("c")
    crow0 = wid * (PER_W // CHUNK)
    rbase = wid * PER_W
    gbufs = (gbuf0, gbuf1)
    vbufs = (vbuf0, vbuf1)
    gsems = (gs0, gs1)
    sgsems = (sg0, sg1)
    svsems = (sv0, sv1)

    @pl.when(lax.axis_index("s") == 0)
    def _stage_shared_table():
        pltpu.sync_copy(table_hbm, table_s)

    pltpu.sync_copy(table_hbm, table_v)
    plsc.subcore_barrier()

    t0 = [table_v[0, pl.ds(k * L, L)] for k in range(NV)]
    t1 = [table_v[1, pl.ds(k * L, L)] for k in range(NV)]
    one = jnp.ones((L,), jnp.int32)
    zero = jnp.zeros((L,), jnp.int32)

    def fill_rows(buf, j):
        def group(g, carry):
            ivec = idx_v[j, pl.ds(g * L, L)]
            w0f = jnp.maximum(zero, one - ivec).astype(jnp.float32)
            w1f = jnp.maximum(zero, one - jnp.abs(ivec - one)).astype(
                jnp.float32)
            for l in range(L):
                a = jnp.full((L,), w0f[l], jnp.float32)
                b = jnp.full((L,), w1f[l], jnp.float32)
                i = g * L + l
                for k in range(NV):
                    buf[i, pl.ds(k * L, L)] = a * t0[k] + b * t1[k]
            return carry
        lax.fori_loop(0, CHUNK // L, group, 0, unroll=2)

    def block(blk, carry):
        pltpu.sync_copy(idx_hbm.at[pl.ds(crow0 + blk * K, K)], idx_v)
        gathers = [None, None]
        gstores = [None, None]
        vstores = [None, None]
        gcnt = 0
        vcnt = 0
        pending = None
        for j in range(K):
            if VALU_CHUNK[j]:
                b = vcnt & 1
                if vstores[b] is not None:
                    vstores[b].wait()
                fill_rows(vbufs[b], j)
                vstores[b] = pltpu.async_copy(
                    vbufs[b],
                    out_hbm.at[pl.ds(rbase + (blk * K + j) * CHUNK, CHUNK)],
                    svsems[b],
                )
                vcnt += 1
            else:
                b = gcnt & 1
                if gstores[b] is not None:
                    gstores[b].wait()
                gathers[b] = pltpu.async_copy(
                    table_s.at[idx_v.at[j]], gbufs[b], gsems[b])
                if pending is not None:
                    pc, pb, pj = pending
                    pc.wait()
                    gstores[pb] = pltpu.async_copy(
                        gbufs[pb],
                        out_hbm.at[pl.ds(rbase + (blk * K + pj) * CHUNK,
                                         CHUNK)],
                        sgsems[pb],
                    )
                pending = (gathers[b], b, j)
                gcnt += 1
        pc, pb, pj = pending
        pc.wait()
        gstores[pb] = pltpu.async_copy(
            gbufs[pb],
            out_hbm.at[pl.ds(rbase + (blk * K + pj) * CHUNK, CHUNK)],
            sgsems[pb],
        )
        for st in gstores + vstores:
            if st is not None:
                st.wait()
        return carry

    lax.fori_loop(0, NBLK, block, 0)


def kernel(inputs, table):
    pad_mask = (jnp.arange(3) != PAD_IDX).astype(table.dtype)[:, None]
    masked_table = table * pad_mask
    idx2d = inputs.reshape(TOTAL // CHUNK, CHUNK)
    out = _sc_embed(idx2d, masked_table)
    return out.reshape(BATCH, SEQ, EMBED)


# speedup vs baseline: 1.7093x; 1.0179x over previous
import functools

import jax
import jax.numpy as jnp
from jax import lax
from jax.experimental import pallas as pl
from jax.experimental.pallas import tpu as pltpu
from jax.experimental.pallas import tpu_sc as plsc

BATCH = 16384
SEQ = 200
EMBED = 128
PAD_IDX = 2
TOTAL = BATCH * SEQ

NC = 2
NS = 16
NW = NC * NS
PER_W = TOTAL // NW
CHUNK = 128
K = 16
NBLK = PER_W // (CHUNK * K)
L = 16
NV = EMBED // L

VALU_CHUNK = tuple(j in (1, 3, 5, 7, 9, 11, 13) for j in range(K))

_mesh = plsc.VectorSubcoreMesh(core_axis_name="c", subcore_axis_name="s")


@functools.partial(
    pl.kernel,
    out_type=jax.ShapeDtypeStruct((TOTAL, EMBED), jnp.float32),
    mesh=_mesh,
    scratch_types=[
        pltpu.VMEM_SHARED((3, EMBED), jnp.float32),
        pltpu.VMEM((3, EMBED), jnp.float32),
        pltpu.VMEM((K, CHUNK), jnp.int32),
        pltpu.VMEM((CHUNK, EMBED), jnp.float32),
        pltpu.VMEM((CHUNK, EMBED), jnp.float32),
        pltpu.VMEM((CHUNK, EMBED), jnp.float32),
        pltpu.VMEM((CHUNK, EMBED), jnp.float32),
        pltpu.SemaphoreType.DMA,
        pltpu.SemaphoreType.DMA,
        pltpu.SemaphoreType.DMA,
        pltpu.SemaphoreType.DMA,
        pltpu.SemaphoreType.DMA,
        pltpu.SemaphoreType.DMA,
    ],
)
def _sc_embed(idx_hbm, table_hbm, out_hbm, table_s, table_v, idx_v,
              gbuf0, gbuf1, vbuf0, vbuf1,
              gs0, gs1, sg0, sg1, sv0, sv1):
    wid = lax.axis_index("s") * NC + lax.axis_index("c")
    crow0 = wid * (PER_W // CHUNK)
    rbase = wid * PER_W
    gbufs = (gbuf0, gbuf1)
    vbufs = (vbuf0, vbuf1)
    gsems = (gs0, gs1)
    sgsems = (sg0, sg1)
    svsems = (sv0, sv1)

    @pl.when(lax.axis_index("s") == 0)
    def _stage_shared_table():
        pltpu.sync_copy(table_hbm, table_s)

    pltpu.sync_copy(table_hbm, table_v)
    plsc.subcore_barrier()

    t0 = [table_v[0, pl.ds(k * L, L)] for k in range(NV)]
    t1 = [table_v[1, pl.ds(k * L, L)] for k in range(NV)]
    one = jnp.ones((L,), jnp.int32)
    zero = jnp.zeros((L,), jnp.int32)

    def fill_rows(buf, j):
        def group(g, carry):
            ivec = idx_v[j, pl.ds(g * L, L)]
            w0f = jnp.maximum(zero, one - ivec).astype(jnp.float32)
            w1f = jnp.maximum(zero, one - jnp.abs(ivec - one)).astype(
                jnp.float32)
            for l in range(L):
                a = jnp.full((L,), w0f[l], jnp.float32)
                b = jnp.full((L,), w1f[l], jnp.float32)
                i = g * L + l
                for k in range(NV):
                    buf[i, pl.ds(k * L, L)] = a * t0[k] + b * t1[k]
            return carry
        lax.fori_loop(0, CHUNK // L, group, 0, unroll=2)

    def block(blk, carry):
        pltpu.sync_copy(idx_hbm.at[pl.ds(crow0 + blk * K, K)], idx_v)
        gathers = [None, None]
        gstores = [None, None]
        vstores = [None, None]
        gcnt = 0
        vcnt = 0
        pending = None
        for j in range(K):
            if VALU_CHUNK[j]:
                b = vcnt & 1
                if vstores[b] is not None:
                    vstores[b].wait()
                fill_rows(vbufs[b], j)
                vstores[b] = pltpu.async_copy(
                    vbufs[b],
                    out_hbm.at[pl.ds(rbase + (blk * K + j) * CHUNK, CHUNK)],
                    svsems[b],
                )
                vcnt += 1
            else:
                b = gcnt & 1
                if gstores[b] is not None:
                    gstores[b].wait()
                gathers[b] = pltpu.async_copy(
                    table_s.at[idx_v.at[j]], gbufs[b], gsems[b])
                if pending is not None:
                    pc, pb, pj = pending
                    pc.wait()
                    gstores[pb] = pltpu.async_copy(
                        gbufs[pb],
                        out_hbm.at[pl.ds(rbase + (blk * K + pj) * CHUNK,
                                         CHUNK)],
                        sgsems[pb],
                    )
                pending = (gathers[b], b, j)
                gcnt += 1
        pc, pb, pj = pending
        pc.wait()
        gstores[pb] = pltpu.async_copy(
            gbufs[pb],
            out_hbm.at[pl.ds(rbase + (blk * K + pj) * CHUNK, CHUNK)],
            sgsems[pb],
        )
        for st in gstores + vstores:
            if st is not None:
                st.wait()
        return carry

    lax.fori_loop(0, NBLK, block, 0)


def kernel(inputs, table):
    pad_mask = (jnp.arange(3) != PAD_IDX).astype(table.dtype)[:, None]
    masked_table = table * pad_mask
    idx2d = inputs.reshape(TOTAL // CHUNK, CHUNK)
    out = _sc_embed(idx2d, masked_table)
    return out.reshape(BATCH, SEQ, EMBED)
